# baseline (speedup 1.0000x reference)
import functools

import jax
import jax.numpy as jnp
from jax import lax
from jax.experimental import pallas as pl
from jax.experimental.pallas import tpu as pltpu
from jax.experimental.pallas import tpu_sc as plsc

EPS_LN = 1e-5

_NC = 2
_NS = 16
_NW = _NC * _NS


def _sc_gather(table2, half_idx, b, l):
    n = half_idx.shape[0]
    d2 = table2.shape[1]
    dt = table2.dtype
    b_per_w = n // _NW
    chunk = 640
    assert b_per_w % chunk == 0

    mesh = plsc.VectorSubcoreMesh(core_axis_name="c", subcore_axis_name="s")

    @functools.partial(
        pl.kernel,
        mesh=mesh,
        out_type=jax.ShapeDtypeStruct((b, l, d2), dt),
        scratch_types=[
            pltpu.VMEM((chunk,), jnp.int32),
            pltpu.VMEM((chunk, d2), dt),
            pltpu.SemaphoreType.DMA,
        ],
    )
    def gather_kernel(table_hbm, idx_hbm, out_hbm, idx_v, rows_v, sem):
        out2 = out_hbm.reshape(n, d2)
        wid = lax.axis_index("s") * _NC + lax.axis_index("c")
        base = wid * b_per_w

        @pl.loop(0, b_per_w, step=chunk)
        def _(off):
            pltpu.sync_copy(idx_hbm.at[pl.ds(base + off, chunk)], idx_v)
            pltpu.async_copy(table_hbm.at[idx_v], rows_v, sem).wait()
            pltpu.sync_copy(rows_v, out2.at[pl.ds(base + off, chunk)])

    return gather_kernel(table2, half_idx)


_PAIR_BO = 4096


def _tp_body(tT_ref, o_ref):
    d = tT_ref.shape[0]
    eye = jnp.eye(d, dtype=jnp.float32)
    a = tT_ref[:, :_PAIR_BO]
    c = tT_ref[:, _PAIR_BO:]
    dn = (((0,), (0,)), ((), ()))
    o_ref[:, :d] = lax.dot_general(a, eye, dn,
                                   preferred_element_type=jnp.float32)
    o_ref[:, d:] = lax.dot_general(c, eye, dn,
                                   preferred_element_type=jnp.float32)


def _tc_pair_table(tableT):
    d, v = tableT.shape
    bo = _PAIR_BO
    grid = (v + 2 * bo - 1) // (2 * bo)
    return pl.pallas_call(
        _tp_body,
        grid=(grid,),
        in_specs=[pl.BlockSpec((d, 2 * bo), lambda i: (0, i))],
        out_specs=pl.BlockSpec((bo, 2 * d), lambda i: (i, 0)),
        out_shape=jax.ShapeDtypeStruct((grid * bo, 2 * d), jnp.float32),
    )(tableT)


def _ln_body(x_ref, par_ref, pos_ref, lab_ref, seg_ref, g_ref, b_ref, o_ref):
    lb = o_ref.shape[0]
    d = o_ref.shape[1]
    eye = jnp.eye(2 * d, dtype=jnp.float32)
    lab = lab_ref[...]
    seg = jnp.where(lab == 0, seg_ref[0:1, :], seg_ref[1:2, :])
    comb_cols = (pos_ref[...] + seg).T
    g_col = g_ref[...].T
    b_col = b_ref[...].T
    for li in range(lb):
        x_l = x_ref[:, li, :]
        xt = lax.dot_general(eye, x_l, (((1,), (1,)), ((), ())),
                             preferred_element_type=jnp.float32)
        par_l = par_ref[li, :]
        tok = jnp.where(par_l[None, :] == 0, xt[:d, :], xt[d:, :])
        emb = tok + comb_cols[:, li:li + 1]
        mean = jnp.mean(emb, axis=0, keepdims=True)
        cen = emb - mean
        var = jnp.mean(cen * cen, axis=0, keepdims=True)
        inv = lax.rsqrt(var + EPS_LN)
        o_ref[li, :, :] = cen * inv * g_col + b_col


def _ln(x2, parT, pos, lab, seg_tab, gamma, beta):
    b, l, d2 = x2.shape
    d = d2 // 2
    lb = 8
    bblk = 512
    lab2 = lab.reshape(l, 1).astype(jnp.int32)
    return pl.pallas_call(
        _ln_body,
        grid=(l // lb, b // bblk),
        in_specs=[
            pl.BlockSpec((bblk, lb, d2), lambda i, j: (j, i, 0)),
            pl.BlockSpec((lb, bblk), lambda i, j: (i, j)),
            pl.BlockSpec((lb, d), lambda i, j: (i, 0)),
            pl.BlockSpec((lb, 1), lambda i, j: (i, 0)),
            pl.BlockSpec((2, d), lambda i, j: (0, 0)),
            pl.BlockSpec((1, d), lambda i, j: (0, 0)),
            pl.BlockSpec((1, d), lambda i, j: (0, 0)),
        ],
        out_specs=pl.BlockSpec((lb, d, bblk), lambda i, j: (i, 0, j)),
        out_shape=jax.ShapeDtypeStruct((l, d, b), jnp.float32),
    )(x2, parT, pos, lab2, seg_tab, gamma.reshape(1, d), beta.reshape(1, d))


def kernel(sequence, segment_label, token_table, position_table, segment_table, gamma, beta):
    b, l = sequence.shape
    d = token_table.shape[1]
    seq32 = sequence.astype(jnp.int32)
    flat = seq32.reshape(-1)
    table2 = token_table.reshape(token_table.shape[0] // 2, 2 * d)
    row_idx = flat >> 1
    halfT = (seq32 & 1).T
    gathered = _sc_gather(table2, row_idx, b, l)
    out_t = _ln(gathered, halfT, position_table[:l],
                segment_label, segment_table, gamma, beta)
    return out_t.transpose(2, 0, 1)


# speedup vs baseline: 1.8790x; 1.8790x over previous
import functools

import jax
import jax.numpy as jnp
from jax import lax
from jax.experimental import pallas as pl
from jax.experimental.pallas import tpu as pltpu
from jax.experimental.pallas import tpu_sc as plsc

EPS_LN = 1e-5

_NC = 2
_NS = 16
_NW = _NC * _NS


def _sc_gather(table2, half_idx, b, l):
    n = half_idx.shape[0]
    d2 = table2.shape[1]
    b_per_w = n // _NW
    chunk = 640
    assert b_per_w % chunk == 0

    mesh = plsc.VectorSubcoreMesh(core_axis_name="c", subcore_axis_name="s")

    @functools.partial(
        pl.kernel,
        mesh=mesh,
        out_type=jax.ShapeDtypeStruct((b, l, d2), jnp.float32),
        scratch_types=[
            pltpu.VMEM((chunk,), jnp.int32),
            pltpu.VMEM((chunk, d2), jnp.float32),
            pltpu.SemaphoreType.DMA,
        ],
    )
    def gather_kernel(table_hbm, idx_hbm, out_hbm, idx_v, rows_v, sem):
        out2 = out_hbm.reshape(n, d2)
        wid = lax.axis_index("s") * _NC + lax.axis_index("c")
        base = wid * b_per_w

        @pl.loop(0, b_per_w, step=chunk)
        def _(off):
            pltpu.sync_copy(idx_hbm.at[pl.ds(base + off, chunk)], idx_v)
            pltpu.async_copy(table_hbm.at[idx_v], rows_v, sem).wait()
            pltpu.sync_copy(rows_v, out2.at[pl.ds(base + off, chunk)])

    return gather_kernel(table2, half_idx)


_PAIR_BO = 8192


def _tp_body(tT_ref, o_ref):
    d = tT_ref.shape[0]
    eye = jnp.eye(d, dtype=jnp.float32)
    a = tT_ref[:, :_PAIR_BO]
    c = tT_ref[:, _PAIR_BO:]
    dn = (((0,), (0,)), ((), ()))
    o_ref[:, :d] = lax.dot_general(a, eye, dn,
                                   preferred_element_type=jnp.float32)
    o_ref[:, d:] = lax.dot_general(c, eye, dn,
                                   preferred_element_type=jnp.float32)


def _tc_pair_table(tableT):
    d, v = tableT.shape
    bo = _PAIR_BO
    grid = (v + 2 * bo - 1) // (2 * bo)
    return pl.pallas_call(
        _tp_body,
        grid=(grid,),
        in_specs=[pl.BlockSpec((d, 2 * bo), lambda i: (0, i))],
        out_specs=pl.BlockSpec((bo, 2 * d), lambda i: (i, 0)),
        out_shape=jax.ShapeDtypeStruct((grid * bo, 2 * d), jnp.float32),
    )(tableT)


def _ln_body(x_ref, par_ref, pos_ref, lab_ref, seg_ref, g_ref, b_ref, o_ref):
    lb = o_ref.shape[0]
    d = o_ref.shape[1]
    eye = jnp.eye(2 * d, dtype=jnp.float32)
    lab = lab_ref[...]
    seg = jnp.where(lab == 0, seg_ref[0:1, :], seg_ref[1:2, :])
    comb_cols = (pos_ref[...] + seg).T
    g_col = g_ref[...].T
    b_col = b_ref[...].T
    for li in range(lb):
        x_l = x_ref[:, li, :]
        xt = lax.dot_general(eye, x_l, (((1,), (1,)), ((), ())),
                             preferred_element_type=jnp.float32)
        par_l = par_ref[li, :]
        tok = jnp.where(par_l[None, :] == 0, xt[:d, :], xt[d:, :])
        emb = tok + comb_cols[:, li:li + 1]
        mean = jnp.mean(emb, axis=0, keepdims=True)
        cen = emb - mean
        var = jnp.mean(cen * cen, axis=0, keepdims=True)
        inv = lax.rsqrt(var + EPS_LN)
        o_ref[li, :, :] = cen * inv * g_col + b_col


def _ln(x2, parT, pos, lab, seg_tab, gamma, beta):
    b, l, d2 = x2.shape
    d = d2 // 2
    lb = 8
    bblk = 1024
    lab2 = lab.reshape(l, 1).astype(jnp.int32)
    return pl.pallas_call(
        _ln_body,
        grid=(l // lb, b // bblk),
        in_specs=[
            pl.BlockSpec((bblk, lb, d2), lambda i, j: (j, i, 0)),
            pl.BlockSpec((lb, bblk), lambda i, j: (i, j)),
            pl.BlockSpec((lb, d), lambda i, j: (i, 0)),
            pl.BlockSpec((lb, 1), lambda i, j: (i, 0)),
            pl.BlockSpec((2, d), lambda i, j: (0, 0)),
            pl.BlockSpec((1, d), lambda i, j: (0, 0)),
            pl.BlockSpec((1, d), lambda i, j: (0, 0)),
        ],
        out_specs=pl.BlockSpec((lb, d, bblk), lambda i, j: (i, 0, j)),
        out_shape=jax.ShapeDtypeStruct((l, d, b), jnp.float32),
    )(x2, parT, pos, lab2, seg_tab, gamma.reshape(1, d), beta.reshape(1, d))


def kernel(sequence, segment_label, token_table, position_table, segment_table, gamma, beta):
    b, l = sequence.shape
    d = token_table.shape[1]
    seq32 = sequence.astype(jnp.int32)
    flat = seq32.reshape(-1)
    table2 = _tc_pair_table(token_table.T)
    row_idx = ((flat >> 14) << 13) | (flat & (_PAIR_BO - 1))
    halfT = ((seq32 >> 13) & 1).T
    gathered = _sc_gather(table2, row_idx, b, l)
    out_t = _ln(gathered, halfT, position_table[:l],
                segment_label, segment_table, gamma, beta)
    return out_t.transpose(2, 0, 1)


# speedup vs baseline: 1.9513x; 1.0385x over previous
import functools

import jax
import jax.numpy as jnp
from jax import lax
from jax.experimental import pallas as pl
from jax.experimental.pallas import tpu as pltpu
from jax.experimental.pallas import tpu_sc as plsc

EPS_LN = 1e-5

_NC = 2
_NS = 16
_NW = _NC * _NS


def _sc_gather(table2, half_idx, b, l):
    n = half_idx.shape[0]
    d2 = table2.shape[1]
    b_per_w = n // _NW
    chunk = 640
    assert b_per_w % chunk == 0

    mesh = plsc.VectorSubcoreMesh(core_axis_name="c", subcore_axis_name="s")

    @functools.partial(
        pl.kernel,
        mesh=mesh,
        out_type=jax.ShapeDtypeStruct((b, l, d2), jnp.float32),
        scratch_types=[
            pltpu.VMEM((chunk,), jnp.int32),
            pltpu.VMEM((chunk, d2), jnp.float32),
            pltpu.SemaphoreType.DMA,
        ],
    )
    def gather_kernel(table_hbm, idx_hbm, out_hbm, idx_v, rows_v, sem):
        out2 = out_hbm.reshape(n, d2)
        wid = lax.axis_index("s") * _NC + lax.axis_index("c")
        base = wid * b_per_w

        @pl.loop(0, b_per_w, step=chunk)
        def _(off):
            pltpu.sync_copy(idx_hbm.at[pl.ds(base + off, chunk)], idx_v)
            pltpu.async_copy(table_hbm.at[idx_v], rows_v, sem).wait()
            pltpu.sync_copy(rows_v, out2.at[pl.ds(base + off, chunk)])

    return gather_kernel(table2, half_idx)


_PAIR_BO = 16384


def _tp_body(tT_ref, o_ref):
    d = tT_ref.shape[0]
    eye = jnp.eye(d, dtype=jnp.float32)
    a = tT_ref[:, :_PAIR_BO]
    c = tT_ref[:, _PAIR_BO:]
    dn = (((0,), (0,)), ((), ()))
    o_ref[:, :d] = lax.dot_general(a, eye, dn,
                                   preferred_element_type=jnp.float32)
    o_ref[:, d:] = lax.dot_general(c, eye, dn,
                                   preferred_element_type=jnp.float32)


def _tc_pair_table(tableT):
    d, v = tableT.shape
    bo = _PAIR_BO
    grid = (v + 2 * bo - 1) // (2 * bo)
    return pl.pallas_call(
        _tp_body,
        grid=(grid,),
        in_specs=[pl.BlockSpec((d, 2 * bo), lambda i: (0, i))],
        out_specs=pl.BlockSpec((bo, 2 * d), lambda i: (i, 0)),
        out_shape=jax.ShapeDtypeStruct((grid * bo, 2 * d), jnp.float32),
    )(tableT)


def _ln_body(x_ref, par_ref, pos_ref, lab_ref, seg_ref, g_ref, b_ref, o_ref):
    lb = o_ref.shape[0]
    d = o_ref.shape[1]
    eye = jnp.eye(2 * d, dtype=jnp.float32)
    lab = lab_ref[...]
    seg = jnp.where(lab == 0, seg_ref[0:1, :], seg_ref[1:2, :])
    comb_cols = (pos_ref[...] + seg).T
    g_col = g_ref[...].T
    b_col = b_ref[...].T
    for li in range(lb):
        x_l = x_ref[:, li, :]
        xt = lax.dot_general(eye, x_l, (((1,), (1,)), ((), ())),
                             preferred_element_type=jnp.float32)
        par_l = par_ref[li, :]
        tok = jnp.where(par_l[None, :] == 0, xt[:d, :], xt[d:, :])
        emb = tok + comb_cols[:, li:li + 1]
        mean = jnp.mean(emb, axis=0, keepdims=True)
        cen = emb - mean
        var = jnp.mean(cen * cen, axis=0, keepdims=True)
        inv = lax.rsqrt(var + EPS_LN)
        o_ref[li, :, :] = cen * inv * g_col + b_col


def _ln(x2, parT, pos, lab, seg_tab, gamma, beta):
    b, l, d2 = x2.shape
    d = d2 // 2
    lb = 8
    bblk = 1024
    lab2 = lab.reshape(l, 1).astype(jnp.int32)
    return pl.pallas_call(
        _ln_body,
        grid=(l // lb, b // bblk),
        in_specs=[
            pl.BlockSpec((bblk, lb, d2), lambda i, j: (j, i, 0)),
            pl.BlockSpec((lb, bblk), lambda i, j: (i, j)),
            pl.BlockSpec((lb, d), lambda i, j: (i, 0)),
            pl.BlockSpec((lb, 1), lambda i, j: (i, 0)),
            pl.BlockSpec((2, d), lambda i, j: (0, 0)),
            pl.BlockSpec((1, d), lambda i, j: (0, 0)),
            pl.BlockSpec((1, d), lambda i, j: (0, 0)),
        ],
        out_specs=pl.BlockSpec((lb, d, bblk), lambda i, j: (i, 0, j)),
        out_shape=jax.ShapeDtypeStruct((l, d, b), jnp.float32),
    )(x2, parT, pos, lab2, seg_tab, gamma.reshape(1, d), beta.reshape(1, d))


def kernel(sequence, segment_label, token_table, position_table, segment_table, gamma, beta):
    b, l = sequence.shape
    d = token_table.shape[1]
    seq32 = sequence.astype(jnp.int32)
    flat = seq32.reshape(-1)
    table2 = _tc_pair_table(token_table.T)
    row_idx = ((flat >> 15) << 14) | (flat & (_PAIR_BO - 1))
    halfT = ((seq32 >> 14) & 1).T
    gathered = _sc_gather(table2, row_idx, b, l)
    out_t = _ln(gathered, halfT, position_table[:l],
                segment_label, segment_table, gamma, beta)
    return out_t.transpose(2, 0, 1)


# speedup vs baseline: 1.9717x; 1.0104x over previous
import functools

import jax
import jax.numpy as jnp
from jax import lax
from jax.experimental import pallas as pl
from jax.experimental.pallas import tpu as pltpu
from jax.experimental.pallas import tpu_sc as plsc

EPS_LN = 1e-5

_NC = 2
_NS = 16
_NW = _NC * _NS


def _sc_gather(table2, half_idx, b, l):
    n = half_idx.shape[0]
    d2 = table2.shape[1]
    b_per_w = n // _NW
    chunk = 400
    nchunk = b_per_w // chunk
    assert b_per_w % chunk == 0

    mesh = plsc.VectorSubcoreMesh(core_axis_name="c", subcore_axis_name="s")

    @functools.partial(
        pl.kernel,
        mesh=mesh,
        out_type=jax.ShapeDtypeStruct((b, l, d2), jnp.float32),
        scratch_types=[
            pltpu.VMEM((chunk,), jnp.int32),
            pltpu.VMEM((chunk,), jnp.int32),
            pltpu.VMEM((chunk, d2), jnp.float32),
            pltpu.VMEM((chunk, d2), jnp.float32),
            pltpu.SemaphoreType.DMA,
            pltpu.SemaphoreType.DMA,
        ],
    )
    def gather_kernel(table_hbm, idx_hbm, out_hbm,
                      idx0, idx1, rows0, rows1, sem0, sem1):
        out2 = out_hbm.reshape(n, d2)
        wid = lax.axis_index("s") * _NC + lax.axis_index("c")
        base = wid * b_per_w
        idxs = (idx0, idx1)
        rows = (rows0, rows1)
        sems = (sem0, sem1)

        pltpu.sync_copy(idx_hbm.at[pl.ds(base, chunk)], idx0)
        g = pltpu.async_copy(table_hbm.at[idx0], rows0, sem0)
        for k in range(nchunk):
            cur = k % 2
            nxt = (k + 1) % 2
            if k + 1 < nchunk:
                off = (k + 1) * chunk
                pltpu.sync_copy(idx_hbm.at[pl.ds(base + off, chunk)], idxs[nxt])
                gn = pltpu.async_copy(table_hbm.at[idxs[nxt]], rows[nxt],
                                      sems[nxt])
            g.wait()
            pltpu.sync_copy(rows[cur], out2.at[pl.ds(base + k * chunk, chunk)])
            if k + 1 < nchunk:
                g = gn

    return gather_kernel(table2, half_idx)


_PAIR_BO = 16384


def _tp_body(tT_ref, o_ref):
    d = tT_ref.shape[0]
    eye = jnp.eye(d, dtype=jnp.float32)
    a = tT_ref[:, :_PAIR_BO]
    c = tT_ref[:, _PAIR_BO:]
    dn = (((0,), (0,)), ((), ()))
    o_ref[:, :d] = lax.dot_general(a, eye, dn,
                                   preferred_element_type=jnp.float32)
    o_ref[:, d:] = lax.dot_general(c, eye, dn,
                                   preferred_element_type=jnp.float32)


def _tc_pair_table(tableT):
    d, v = tableT.shape
    bo = _PAIR_BO
    grid = (v + 2 * bo - 1) // (2 * bo)
    return pl.pallas_call(
        _tp_body,
        grid=(grid,),
        in_specs=[pl.BlockSpec((d, 2 * bo), lambda i: (0, i))],
        out_specs=pl.BlockSpec((bo, 2 * d), lambda i: (i, 0)),
        out_shape=jax.ShapeDtypeStruct((grid * bo, 2 * d), jnp.float32),
    )(tableT)


def _ln_body(x_ref, par_ref, pos_ref, lab_ref, seg_ref, g_ref, b_ref, o_ref):
    lb = o_ref.shape[0]
    d = o_ref.shape[1]
    eye = jnp.eye(2 * d, dtype=jnp.float32)
    lab = lab_ref[...]
    seg = jnp.where(lab == 0, seg_ref[0:1, :], seg_ref[1:2, :])
    comb_cols = (pos_ref[...] + seg).T
    g_col = g_ref[...].T
    b_col = b_ref[...].T
    for li in range(lb):
        x_l = x_ref[:, li, :]
        xt = lax.dot_general(eye, x_l, (((1,), (1,)), ((), ())),
                             preferred_element_type=jnp.float32)
        par_l = par_ref[li, :]
        tok = jnp.where(par_l[None, :] == 0, xt[:d, :], xt[d:, :])
        emb = tok + comb_cols[:, li:li + 1]
        mean = jnp.mean(emb, axis=0, keepdims=True)
        cen = emb - mean
        var = jnp.mean(cen * cen, axis=0, keepdims=True)
        inv = lax.rsqrt(var + EPS_LN)
        o_ref[li, :, :] = cen * inv * g_col + b_col


def _ln(x2, parT, pos, lab, seg_tab, gamma, beta):
    b, l, d2 = x2.shape
    d = d2 // 2
    lb = 8
    bblk = 1024
    lab2 = lab.reshape(l, 1).astype(jnp.int32)
    return pl.pallas_call(
        _ln_body,
        grid=(l // lb, b // bblk),
        in_specs=[
            pl.BlockSpec((bblk, lb, d2), lambda i, j: (j, i, 0)),
            pl.BlockSpec((lb, bblk), lambda i, j: (i, j)),
            pl.BlockSpec((lb, d), lambda i, j: (i, 0)),
            pl.BlockSpec((lb, 1), lambda i, j: (i, 0)),
            pl.BlockSpec((2, d), lambda i, j: (0, 0)),
            pl.BlockSpec((1, d), lambda i, j: (0, 0)),
            pl.BlockSpec((1, d), lambda i, j: (0, 0)),
        ],
        out_specs=pl.BlockSpec((lb, d, bblk), lambda i, j: (i, 0, j)),
        out_shape=jax.ShapeDtypeStruct((l, d, b), jnp.float32),
    )(x2, parT, pos, lab2, seg_tab, gamma.reshape(1, d), beta.reshape(1, d))


def kernel(sequence, segment_label, token_table, position_table, segment_table, gamma, beta):
    b, l = sequence.shape
    d = token_table.shape[1]
    seq32 = sequence.astype(jnp.int32)
    flat = seq32.reshape(-1)
    table2 = _tc_pair_table(token_table.T)
    row_idx = ((flat >> 15) << 14) | (flat & (_PAIR_BO - 1))
    halfT = ((seq32 >> 14) & 1).T
    gathered = _sc_gather(table2, row_idx, b, l)
    out_t = _ln(gathered, halfT, position_table[:l],
                segment_label, segment_table, gamma, beta)
    return out_t.transpose(2, 0, 1)


# speedup vs baseline: 1.9829x; 1.0057x over previous
import functools

import jax
import jax.numpy as jnp
from jax import lax
from jax.experimental import pallas as pl
from jax.experimental.pallas import tpu as pltpu
from jax.experimental.pallas import tpu_sc as plsc

EPS_LN = 1e-5

_NC = 2
_NS = 16
_NW = _NC * _NS


def _sc_gather(table2, half_idx, b, l):
    n = half_idx.shape[0]
    d2 = table2.shape[1]
    b_per_w = n // _NW
    chunk = 400
    nchunk = b_per_w // chunk
    assert b_per_w % chunk == 0

    mesh = plsc.VectorSubcoreMesh(core_axis_name="c", subcore_axis_name="s")

    @functools.partial(
        pl.kernel,
        mesh=mesh,
        out_type=jax.ShapeDtypeStruct((b, l, d2), jnp.float32),
        scratch_types=[
            pltpu.VMEM((chunk,), jnp.int32),
            pltpu.VMEM((chunk,), jnp.int32),
            pltpu.VMEM((chunk, d2), jnp.float32),
            pltpu.VMEM((chunk, d2), jnp.float32),
            pltpu.SemaphoreType.DMA,
            pltpu.SemaphoreType.DMA,
            pltpu.SemaphoreType.DMA,
            pltpu.SemaphoreType.DMA,
        ],
    )
    def gather_kernel(table_hbm, idx_hbm, out_hbm,
                      idx0, idx1, rows0, rows1, sem0, sem1, wsem0, wsem1):
        out2 = out_hbm.reshape(n, d2)
        wid = lax.axis_index("s") * _NC + lax.axis_index("c")
        base = wid * b_per_w
        idxs = (idx0, idx1)
        rows = (rows0, rows1)
        sems = (sem0, sem1)
        wsems = (wsem0, wsem1)

        pltpu.sync_copy(idx_hbm.at[pl.ds(base, chunk)], idx0)
        g = pltpu.async_copy(table_hbm.at[idx0], rows0, sem0)
        w = (None, None)
        for k in range(nchunk):
            cur = k % 2
            nxt = (k + 1) % 2
            if k + 1 < nchunk:
                off = (k + 1) * chunk
                pltpu.sync_copy(idx_hbm.at[pl.ds(base + off, chunk)], idxs[nxt])
                if w[nxt] is not None:
                    w[nxt].wait()
                    w = (w[0], None) if nxt else (None, w[1])
                gn = pltpu.async_copy(table_hbm.at[idxs[nxt]], rows[nxt],
                                      sems[nxt])
            g.wait()
            wk = pltpu.async_copy(rows[cur], out2.at[pl.ds(base + k * chunk,
                                                           chunk)], wsems[cur])
            w = (w[0], wk) if cur else (wk, w[1])
            if k + 1 < nchunk:
                g = gn
        for wk in w:
            if wk is not None:
                wk.wait()

    return gather_kernel(table2, half_idx)


_PAIR_BO = 16384


def _tp_body(tT_ref, o_ref):
    d = tT_ref.shape[0]
    eye = jnp.eye(d, dtype=jnp.float32)
    a = tT_ref[:, :_PAIR_BO]
    c = tT_ref[:, _PAIR_BO:]
    dn = (((0,), (0,)), ((), ()))
    o_ref[:, :d] = lax.dot_general(a, eye, dn,
                                   preferred_element_type=jnp.float32)
    o_ref[:, d:] = lax.dot_general(c, eye, dn,
                                   preferred_element_type=jnp.float32)


def _tc_pair_table(tableT):
    d, v = tableT.shape
    bo = _PAIR_BO
    grid = (v + 2 * bo - 1) // (2 * bo)
    return pl.pallas_call(
        _tp_body,
        grid=(grid,),
        in_specs=[pl.BlockSpec((d, 2 * bo), lambda i: (0, i))],
        out_specs=pl.BlockSpec((bo, 2 * d), lambda i: (i, 0)),
        out_shape=jax.ShapeDtypeStruct((grid * bo, 2 * d), jnp.float32),
    )(tableT)


def _ln_body(x_ref, par_ref, pos_ref, lab_ref, seg_ref, g_ref, b_ref, o_ref):
    lb = o_ref.shape[0]
    d = o_ref.shape[1]
    eye = jnp.eye(2 * d, dtype=jnp.float32)
    lab = lab_ref[...]
    seg = jnp.where(lab == 0, seg_ref[0:1, :], seg_ref[1:2, :])
    comb_cols = (pos_ref[...] + seg).T
    g_col = g_ref[...].T
    b_col = b_ref[...].T
    for li in range(lb):
        x_l = x_ref[:, li, :]
        xt = lax.dot_general(eye, x_l, (((1,), (1,)), ((), ())),
                             preferred_element_type=jnp.float32)
        par_l = par_ref[li, :]
        tok = jnp.where(par_l[None, :] == 0, xt[:d, :], xt[d:, :])
        emb = tok + comb_cols[:, li:li + 1]
        mean = jnp.mean(emb, axis=0, keepdims=True)
        cen = emb - mean
        var = jnp.mean(cen * cen, axis=0, keepdims=True)
        inv = lax.rsqrt(var + EPS_LN)
        o_ref[li, :, :] = cen * inv * g_col + b_col


def _ln(x2, parT, pos, lab, seg_tab, gamma, beta):
    b, l, d2 = x2.shape
    d = d2 // 2
    lb = 8
    bblk = 1024
    lab2 = lab.reshape(l, 1).astype(jnp.int32)
    return pl.pallas_call(
        _ln_body,
        grid=(l // lb, b // bblk),
        in_specs=[
            pl.BlockSpec((bblk, lb, d2), lambda i, j: (j, i, 0)),
            pl.BlockSpec((lb, bblk), lambda i, j: (i, j)),
            pl.BlockSpec((lb, d), lambda i, j: (i, 0)),
            pl.BlockSpec((lb, 1), lambda i, j: (i, 0)),
            pl.BlockSpec((2, d), lambda i, j: (0, 0)),
            pl.BlockSpec((1, d), lambda i, j: (0, 0)),
            pl.BlockSpec((1, d), lambda i, j: (0, 0)),
        ],
        out_specs=pl.BlockSpec((lb, d, bblk), lambda i, j: (i, 0, j)),
        out_shape=jax.ShapeDtypeStruct((l, d, b), jnp.float32),
    )(x2, parT, pos, lab2, seg_tab, gamma.reshape(1, d), beta.reshape(1, d))


def kernel(sequence, segment_label, token_table, position_table, segment_table, gamma, beta):
    b, l = sequence.shape
    d = token_table.shape[1]
    seq32 = sequence.astype(jnp.int32)
    flat = seq32.reshape(-1)
    table2 = _tc_pair_table(token_table.T)
    row_idx = ((flat >> 15) << 14) | (flat & (_PAIR_BO - 1))
    halfT = ((seq32 >> 14) & 1).T
    gathered = _sc_gather(table2, row_idx, b, l)
    out_t = _ln(gathered, halfT, position_table[:l],
                segment_label, segment_table, gamma, beta)
    return out_t.transpose(2, 0, 1)
